# baseline (speedup 1.0000x reference)
import jax
import jax.numpy as jnp
from jax import lax
from jax.experimental import pallas as pl
from jax.experimental.pallas import tpu as pltpu
from jax.experimental.pallas import tpu_sc as plsc

DIM = 50
DIM_PAD = 56
NC, NS = 2, 16
NW = NC * NS
CHUNK = 512
TAIL = 34


def _gather_body(table_hbm, idx_hbm, out_hbm,
                 idx_v0, idx_v1, raw_v0, raw_v1, pck_v0, pck_v1,
                 sem_g0, sem_g1, sem_o0, sem_o1):
    wid = lax.axis_index("s") * NC + lax.axis_index("c")
    n = idx_hbm.shape[0]
    b_per_w = n // NW
    n_chunks = b_per_w // CHUNK
    base = wid * b_per_w
    idx_bufs = (idx_v0, idx_v1)
    raw_bufs = (raw_v0, raw_v1)
    pck_bufs = (pck_v0, pck_v1)
    sem_g = (sem_g0, sem_g1)
    sem_o = (sem_o0, sem_o1)
    lanes = lax.iota(jnp.int32, 16) + TAIL

    def start_gather(g, b):
        off = base + g * CHUNK
        pltpu.sync_copy(idx_hbm.at[pl.ds(off, CHUNK)], idx_bufs[b])
        pltpu.make_async_copy(
            table_hbm.at[idx_bufs[b]], raw_bufs[b], sem_g[b]).start()

    def gather_wait(b):
        pltpu.make_async_copy(
            table_hbm.at[idx_bufs[b]], raw_bufs[b], sem_g[b]).wait()

    def out_copy(g, b):
        off = base + g * CHUNK
        return pltpu.make_async_copy(
            pck_bufs[b], out_hbm.at[pl.ds(off, CHUNK)], sem_o[b])

    def depad(b):
        def rows(r2, carry):
            for u in range(4):
                r = r2 * 4 + u
                for c in (0, 16, 32, TAIL):
                    pck_bufs[b][r, pl.ds(c, 16)] = raw_bufs[b][r, pl.ds(c, 16)]
            return carry

        lax.fori_loop(0, CHUNK // 4, rows, 0)

    start_gather(0, 0)

    def body(g, carry):
        b = lax.rem(g, 2)
        for bb in (0, 1):
            @pl.when(b == bb)
            def _():
                gather_wait(bb)

                @pl.when(g + 1 < n_chunks)
                def _():
                    start_gather(g + 1, 1 - bb)
                depad(bb)

                @pl.when(g >= 2)
                def _():
                    out_copy(g - 2, bb).wait()
                out_copy(g, bb).start()
        return carry

    lax.fori_loop(0, n_chunks, body, 0)
    out_copy(n_chunks - 2, n_chunks % 2).wait()
    out_copy(n_chunks - 1, 1 - n_chunks % 2).wait()


def kernel(x, table):
    B, L = x.shape
    n = B * L
    idx = x.reshape(n)
    table_p = jnp.pad(table, ((0, 0), (0, DIM_PAD - DIM)))
    mesh = plsc.VectorSubcoreMesh(
        core_axis_name="c", subcore_axis_name="s",
        num_cores=NC, num_subcores=NS)
    out = pl.kernel(
        _gather_body,
        out_type=jax.ShapeDtypeStruct((n, DIM), jnp.float32),
        mesh=mesh,
        scratch_types=[
            pltpu.VMEM((CHUNK,), jnp.int32),
            pltpu.VMEM((CHUNK,), jnp.int32),
            pltpu.VMEM((CHUNK, DIM_PAD), jnp.float32),
            pltpu.VMEM((CHUNK, DIM_PAD), jnp.float32),
            pltpu.VMEM((CHUNK, DIM), jnp.float32),
            pltpu.VMEM((CHUNK, DIM), jnp.float32),
            pltpu.SemaphoreType.DMA,
            pltpu.SemaphoreType.DMA,
            pltpu.SemaphoreType.DMA,
            pltpu.SemaphoreType.DMA,
        ],
        compiler_params=pltpu.CompilerParams(
            use_tc_tiling_on_sc=False, needs_layout_passes=False),
    )(table_p, idx)
    return out.reshape(B, L, DIM)
